# baseline (speedup 1.0000x reference)
import functools

import jax
import jax.numpy as jnp
from jax import lax
from jax.experimental import pallas as pl
from jax.experimental.pallas import tpu as pltpu
from jax.experimental.pallas import tpu_sc as plsc


@functools.lru_cache(maxsize=None)
def _build(BS, S, D2, NC, NS):
    NW = NC * NS
    assert BS % NW == 0 and S % (BS // NW) == 0 and D2 == 128
    b_per_w = BS // NW
    p_per_w = b_per_w // 2
    mesh = plsc.VectorSubcoreMesh(core_axis_name="c", subcore_axis_name="s")

    @functools.partial(
        pl.kernel,
        mesh=mesh,
        out_type=jax.ShapeDtypeStruct((BS, D2), jnp.float32),
        scratch_types=[
            pltpu.VMEM((b_per_w,), jnp.int32),
            pltpu.VMEM((b_per_w, D2), jnp.float32),
            pltpu.VMEM((p_per_w, D2), jnp.float32),
            pltpu.SemaphoreType.DMA,
        ],
    )
    def emb_kernel(idx_hbm, table_hbm, pos_hbm, out_hbm, idx_v, rows_v, pos_v, sem):
        wid = lax.axis_index("s") * NC + lax.axis_index("c")
        base = wid * b_per_w
        pltpu.sync_copy(idx_hbm.at[pl.ds(base, b_per_w)], idx_v)
        gather = pltpu.async_copy(table_hbm.at[idx_v], rows_v, sem)
        pbase = lax.div(lax.rem(base, S), 2)
        pltpu.sync_copy(pos_hbm.at[pl.ds(pbase, p_per_w)], pos_v)
        gather.wait()

        def add_pair(r2, carry):
            row0 = 2 * r2
            for h in range(2):
                for c in range(D2 // 2 // 16):
                    chunk = pos_v[r2, pl.ds(h * 64 + c * 16, 16)]
                    plsc.addupdate(rows_v.at[row0 + h, pl.ds(c * 16, 16)], chunk)
                    plsc.addupdate(rows_v.at[row0 + h, pl.ds(64 + c * 16, 16)], chunk)
            return carry

        lax.fori_loop(0, p_per_w, add_pair, 0)
        pltpu.sync_copy(rows_v, out_hbm.at[pl.ds(base, b_per_w)])

    return emb_kernel


def kernel(X, tok_embd, pos_embd):
    B, S = X.shape
    V, D = tok_embd.shape
    BS = B * S
    try:
        info = plsc.get_sparse_core_info()
        NC, NS = info.num_cores, info.num_subcores
    except Exception:
        NC, NS = 2, 16
    xf = X.reshape(BS).astype(jnp.int32)
    table2 = tok_embd.reshape(V // 2, 2 * D)
    pos2 = pos_embd.reshape(S // 2, 2 * D)
    fn = _build(BS, S, 2 * D, NC, NS)
    out2 = fn(xf >> 1, table2, pos2)
    odd = (xf & 1)[:, None].astype(bool)
    out = jnp.where(odd, out2[:, D:], out2[:, :D])
    return out.reshape(B, S, D)
